# baseline (speedup 1.0000x reference)
import functools

import jax
import jax.numpy as jnp
from jax import lax
from jax.experimental import pallas as pl
from jax.experimental.pallas import tpu as pltpu
from jax.experimental.pallas import tpu_sc as plsc

N_NODES = 10000
N_EDGES = 320000
C = 128

NROWS = 10240
GROW = 10000
CHUNK = 128
N_WORKERS = 32
CHUNKS_PER_W = 80
E_PAD = N_WORKERS * CHUNKS_PER_W * CHUNK
CF = 116
CS = 44
E_ALLOC = (16 * CF + 16 * CS + (CF - CS)) * CHUNK
ROWS_PER_TILE = NROWS // 16

_mesh = plsc.VectorSubcoreMesh(core_axis_name="c", subcore_axis_name="s")


def _fill(ref, rows, width, value):
    def body(i, _):
        for l in range(width // 16):
            ref[i, pl.ds(l * 16, 16)] = jnp.full((16,), value, jnp.float32)
        return 0
    lax.fori_loop(0, rows, body, 0)


DROWS = NROWS // CHUNK


@functools.partial(
    pl.kernel,
    out_type=jax.ShapeDtypeStruct((2 * DROWS, CHUNK), jnp.float32),
    mesh=_mesh,
    compiler_params=pltpu.CompilerParams(needs_layout_passes=False),
    scratch_types=[
        pltpu.VMEM((CHUNKS_PER_W, CHUNK), jnp.int32),
        pltpu.VMEM((DROWS, CHUNK), jnp.float32),
        pltpu.VMEM((DROWS,), jnp.int32),
        pltpu.VMEM((8, CHUNK), jnp.float32),
        pltpu.VMEM_SHARED((DROWS, CHUNK), jnp.float32),
    ],
)
def _sc_degree(dst_hbm, out_hbm, didx, local, rowidx, zrows, deg_sp):
    c = lax.axis_index("c")
    s = lax.axis_index("s")
    wid = c * 16 + s
    _fill(zrows, 8, CHUNK, 0.0)
    @pl.when(s < DROWS // 8)
    def _():
        pltpu.sync_copy(zrows, deg_sp.at[pl.ds(s * 8, 8)])
    ones16 = jnp.ones((16,), jnp.float32)

    def zbody(k, _):
        for l in range(8):
            local[k, pl.ds(l * 16, 16)] = jnp.zeros((16,), jnp.float32)
        return 0
    lax.fori_loop(0, DROWS, zbody, 0)
    def ibody(k, _):
        rowidx[pl.ds(k * 16, 16)] = lax.iota(jnp.int32, 16) + k * 16
        return 0
    lax.fori_loop(0, DROWS // 16, ibody, 0)

    pltpu.sync_copy(dst_hbm.at[pl.ds(wid * CHUNKS_PER_W, CHUNKS_PER_W)], didx)

    def body(j, _):
        for g in range(8):
            w = didx[j, pl.ds(g * 16, 16)]
            v = lax.shift_right_logical(w, 16)
            r = lax.shift_right_logical(v, 7)
            cc = lax.bitwise_and(v, 127)
            plsc.addupdate_scatter(local, [r, cc], ones16)
        return 0
    lax.fori_loop(0, CHUNKS_PER_W, body, 0)
    plsc.subcore_barrier()
    pltpu.sync_copy(local, deg_sp.at[rowidx], add=True)
    plsc.subcore_barrier()
    @pl.when(s < DROWS // 8)
    def _():
        pltpu.sync_copy(
            deg_sp.at[pl.ds(s * 8, 8)],
            out_hbm.at[pl.ds(c * DROWS + s * 8, 8)],
        )


@functools.partial(
    pl.kernel,
    out_type=jax.ShapeDtypeStruct((2 * NROWS, C), jnp.float32),
    mesh=_mesh,
    scratch_types=[
        pltpu.VMEM((CF * CHUNK,), jnp.int32),
        pltpu.VMEM((CHUNK,), jnp.int32),
        pltpu.VMEM((CHUNK,), jnp.int32),
        pltpu.VMEM((CHUNK,), jnp.int32),
        pltpu.VMEM((CHUNK,), jnp.int32),
        pltpu.VMEM((CHUNK, C), jnp.float32),
        pltpu.VMEM((CHUNK, C), jnp.float32),
        pltpu.VMEM_SHARED((NROWS, C), jnp.float32),
        pltpu.SemaphoreType.DMA,
        pltpu.SemaphoreType.DMA,
        pltpu.SemaphoreType.DMA,
        pltpu.SemaphoreType.DMA,
    ],
)
def _sc_segsum(y_hbm, pk_hbm, out_hbm, pidx, ia0, ia1, id0, id1, msg0, msg1,
               acc_sp, gsem0, gsem1, ssem0, ssem1):
    c = lax.axis_index("c")
    s = lax.axis_index("s")
    nch = jnp.where(c == 0, CF, CS)
    base = jnp.where(c == 0, s * CF, 16 * CF + s * CS)
    _fill(msg0, CHUNK, C, 0.0)
    row0 = s * ROWS_PER_TILE
    def zero_body(k, _):
        pltpu.sync_copy(msg0, acc_sp.at[pl.ds(row0 + k * CHUNK, CHUNK)])
        return 0
    lax.fori_loop(0, ROWS_PER_TILE // CHUNK, zero_body, 0)
    pltpu.sync_copy(pk_hbm.at[pl.ds(base * CHUNK, CF * CHUNK)], pidx)
    plsc.subcore_barrier()

    def unpack(ch, ia, idd):
        for g in range(C // 16):
            w = pidx[pl.ds(ch * CHUNK + g * 16, 16)]
            ia[pl.ds(g * 16, 16)] = lax.bitwise_and(w, 0xFFFF)
            idd[pl.ds(g * 16, 16)] = lax.shift_right_logical(w, 16)

    def wait_gather(buf, sem):
        pltpu.make_async_copy(y_hbm.at[ia0], buf, sem).wait()

    def wait_scat(buf, sem):
        pltpu.make_async_copy(buf, acc_sp.at[id0], sem).wait()

    unpack(0, ia0, id0)
    pltpu.async_copy(y_hbm.at[ia0], msg0, gsem0)
    unpack(1, ia1, id1)
    pltpu.async_copy(y_hbm.at[ia1], msg1, gsem1)
    def body(j, _):
        a = 2 * j
        b = 2 * j + 1
        wait_gather(msg0, gsem0)
        pltpu.async_copy(msg0, acc_sp.at[id0], ssem0, add=True)
        wait_gather(msg1, gsem1)
        pltpu.async_copy(msg1, acc_sp.at[id1], ssem1, add=True)
        wait_scat(msg0, ssem0)
        unpack(a + 2, ia0, id0)
        pltpu.async_copy(y_hbm.at[ia0], msg0, gsem0)
        wait_scat(msg1, ssem1)
        unpack(b + 2, ia1, id1)
        pltpu.async_copy(y_hbm.at[ia1], msg1, gsem1)
        return 0
    lax.fori_loop(0, (nch - 2) // 2, body, 0)
    wait_gather(msg0, gsem0)
    pltpu.async_copy(msg0, acc_sp.at[id0], ssem0, add=True)
    wait_gather(msg1, gsem1)
    pltpu.async_copy(msg1, acc_sp.at[id1], ssem1, add=True)
    wait_scat(msg0, ssem0)
    wait_scat(msg1, ssem1)
    plsc.subcore_barrier()
    pltpu.sync_copy(
        acc_sp.at[pl.ds(row0, ROWS_PER_TILE)],
        out_hbm.at[pl.ds(c * NROWS + row0, ROWS_PER_TILE)],
    )


_BS = 1024
_GRID = NROWS // _BS


def _tc1_body(d0_ref, d1_ref, x_ref, w_ref, y_ref, dinv_ref):
    deg = d0_ref[...] + d1_ref[...] + 1.0
    dinv8 = lax.rsqrt(deg)
    nrow = _BS // CHUNK
    i0 = lax.broadcasted_iota(jnp.int32, (_BS, nrow), 0)
    i1 = lax.broadcasted_iota(jnp.int32, (_BS, nrow), 1)
    rowsel = (i0 // CHUNK == i1).astype(jnp.float32)
    t = jnp.dot(rowsel, dinv8, preferred_element_type=jnp.float32)
    j0 = lax.broadcasted_iota(jnp.int32, (_BS, CHUNK), 0)
    j1 = lax.broadcasted_iota(jnp.int32, (_BS, CHUNK), 1)
    lanemask = (j1 == j0 % CHUNK).astype(jnp.float32)
    dinv = jnp.sum(t * lanemask, axis=1, keepdims=True)
    xw = jnp.dot(x_ref[...], w_ref[...], preferred_element_type=jnp.float32)
    y_ref[...] = xw * dinv
    dinv_ref[...] = jnp.broadcast_to(dinv, (_BS, 16))


def _tc2_body(p0_ref, p1_ref, y1_ref, dinv_ref, w_ref, b1_ref, y2_ref):
    dv = dinv_ref[:, 0:1]
    agg = p0_ref[...] + p1_ref[...] + y1_ref[...]
    h = jnp.maximum(agg * dv + b1_ref[...], 0.0)
    y2_ref[...] = jnp.dot(h, w_ref[...], preferred_element_type=jnp.float32) * dv


def _tc3_body(q0_ref, q1_ref, y2_ref, dinv_ref, b2_ref, out_ref):
    dv = dinv_ref[:, 0:1]
    out_ref[...] = (q0_ref[...] + q1_ref[...] + y2_ref[...]) * dv + b2_ref[...]


def _rows(i):
    return (i, 0)


def _rows_hi(i):
    return (i + _GRID, 0)


def _rep(i):
    return (0, 0)


def kernel(x, edge_index, W1, b1, W2, b2):
    ei = edge_index.astype(jnp.int32)
    src = jnp.concatenate(
        [ei[0], jnp.zeros((E_ALLOC - N_EDGES,), jnp.int32)])
    dst = jnp.concatenate(
        [ei[1], jnp.full((E_ALLOC - N_EDGES,), GROW, jnp.int32)])
    packed = src | (dst << 16)
    packed2d = packed[:E_PAD].reshape(-1, CHUNK)
    x_pad = jnp.concatenate(
        [x, jnp.zeros((NROWS - N_NODES, C), jnp.float32)], axis=0)
    b1r = b1.reshape(1, C)
    b2r = b2.reshape(1, C)

    degp = _sc_degree(packed2d)

    y1, dinv = pl.pallas_call(
        _tc1_body,
        grid=(_GRID,),
        in_specs=[
            pl.BlockSpec((_BS // CHUNK, CHUNK), _rows),
            pl.BlockSpec((_BS // CHUNK, CHUNK), _rows_hi),
            pl.BlockSpec((_BS, C), _rows),
            pl.BlockSpec((C, C), _rep),
        ],
        out_specs=[
            pl.BlockSpec((_BS, C), _rows),
            pl.BlockSpec((_BS, 16), _rows),
        ],
        out_shape=[
            jax.ShapeDtypeStruct((NROWS, C), jnp.float32),
            jax.ShapeDtypeStruct((NROWS, 16), jnp.float32),
        ],
    )(degp, degp, x_pad, W1)

    p = _sc_segsum(y1, packed)

    y2 = pl.pallas_call(
        _tc2_body,
        grid=(_GRID,),
        in_specs=[
            pl.BlockSpec((_BS, C), _rows),
            pl.BlockSpec((_BS, C), _rows_hi),
            pl.BlockSpec((_BS, C), _rows),
            pl.BlockSpec((_BS, 16), _rows),
            pl.BlockSpec((C, C), _rep),
            pl.BlockSpec((1, C), _rep),
        ],
        out_specs=pl.BlockSpec((_BS, C), _rows),
        out_shape=jax.ShapeDtypeStruct((NROWS, C), jnp.float32),
    )(p, p, y1, dinv, W2, b1r)

    q = _sc_segsum(y2, packed)

    out = pl.pallas_call(
        _tc3_body,
        grid=(_GRID,),
        in_specs=[
            pl.BlockSpec((_BS, C), _rows),
            pl.BlockSpec((_BS, C), _rows_hi),
            pl.BlockSpec((_BS, C), _rows),
            pl.BlockSpec((_BS, 16), _rows),
            pl.BlockSpec((1, C), _rep),
        ],
        out_specs=pl.BlockSpec((_BS, C), _rows),
        out_shape=jax.ShapeDtypeStruct((NROWS, C), jnp.float32),
    )(q, q, y2, dinv, b2r)

    return out[:N_NODES]


# speedup vs baseline: 1.1130x; 1.1130x over previous
import functools

import jax
import jax.numpy as jnp
from jax import lax
from jax.experimental import pallas as pl
from jax.experimental.pallas import tpu as pltpu
from jax.experimental.pallas import tpu_sc as plsc

N_NODES = 10000
N_EDGES = 320000
C = 128

NROWS = 10240
GROW = 10000
CHUNK = 128
N_WORKERS = 32
CHUNKS_PER_W = 80
E_PAD = N_WORKERS * CHUNKS_PER_W * CHUNK
CF = CHUNKS_PER_W
E_ALLOC = E_PAD
ROWS_PER_TILE = NROWS // 16

_mesh = plsc.VectorSubcoreMesh(core_axis_name="c", subcore_axis_name="s")


def _fill(ref, rows, width, value):
    def body(i, _):
        for l in range(width // 16):
            ref[i, pl.ds(l * 16, 16)] = jnp.full((16,), value, jnp.float32)
        return 0
    lax.fori_loop(0, rows, body, 0)


DROWS = NROWS // CHUNK


@functools.partial(
    pl.kernel,
    out_type=jax.ShapeDtypeStruct((2 * DROWS, CHUNK), jnp.float32),
    mesh=_mesh,
    compiler_params=pltpu.CompilerParams(needs_layout_passes=False),
    scratch_types=[
        pltpu.VMEM((CHUNKS_PER_W, CHUNK), jnp.int32),
        pltpu.VMEM((DROWS, CHUNK), jnp.float32),
        pltpu.VMEM((DROWS,), jnp.int32),
        pltpu.VMEM((8, CHUNK), jnp.float32),
        pltpu.VMEM_SHARED((DROWS, CHUNK), jnp.float32),
    ],
)
def _sc_degree(dst_hbm, out_hbm, didx, local, rowidx, zrows, deg_sp):
    c = lax.axis_index("c")
    s = lax.axis_index("s")
    wid = c * 16 + s
    _fill(zrows, 8, CHUNK, 0.0)
    @pl.when(s < DROWS // 8)
    def _():
        pltpu.sync_copy(zrows, deg_sp.at[pl.ds(s * 8, 8)])
    ones16 = jnp.ones((16,), jnp.float32)

    def zbody(k, _):
        for l in range(8):
            local[k, pl.ds(l * 16, 16)] = jnp.zeros((16,), jnp.float32)
        return 0
    lax.fori_loop(0, DROWS, zbody, 0)
    def ibody(k, _):
        rowidx[pl.ds(k * 16, 16)] = lax.iota(jnp.int32, 16) + k * 16
        return 0
    lax.fori_loop(0, DROWS // 16, ibody, 0)

    pltpu.sync_copy(dst_hbm.at[pl.ds(wid * CHUNKS_PER_W, CHUNKS_PER_W)], didx)

    def body(j, _):
        for g in range(8):
            w = didx[j, pl.ds(g * 16, 16)]
            v = lax.shift_right_logical(w, 16)
            r = lax.shift_right_logical(v, 7)
            cc = lax.bitwise_and(v, 127)
            plsc.addupdate_scatter(local, [r, cc], ones16)
        return 0
    lax.fori_loop(0, CHUNKS_PER_W, body, 0)
    plsc.subcore_barrier()
    pltpu.sync_copy(local, deg_sp.at[rowidx], add=True)
    plsc.subcore_barrier()
    @pl.when(s < DROWS // 8)
    def _():
        pltpu.sync_copy(
            deg_sp.at[pl.ds(s * 8, 8)],
            out_hbm.at[pl.ds(c * DROWS + s * 8, 8)],
        )


@functools.partial(
    pl.kernel,
    out_type=jax.ShapeDtypeStruct((2 * NROWS, C), jnp.float32),
    mesh=_mesh,
    scratch_types=[
        pltpu.VMEM((CF * CHUNK,), jnp.int32),
        pltpu.VMEM((CHUNK,), jnp.int32),
        pltpu.VMEM((CHUNK,), jnp.int32),
        pltpu.VMEM((CHUNK,), jnp.int32),
        pltpu.VMEM((CHUNK,), jnp.int32),
        pltpu.VMEM((CHUNK, C), jnp.float32),
        pltpu.VMEM((CHUNK, C), jnp.float32),
        pltpu.VMEM_SHARED((NROWS, C), jnp.float32),
        pltpu.SemaphoreType.DMA,
        pltpu.SemaphoreType.DMA,
        pltpu.SemaphoreType.DMA,
        pltpu.SemaphoreType.DMA,
    ],
)
def _sc_segsum(y_hbm, pk_hbm, out_hbm, pidx, ia0, ia1, id0, id1, msg0, msg1,
               acc_sp, gsem0, gsem1, ssem0, ssem1):
    c = lax.axis_index("c")
    s = lax.axis_index("s")
    base = (c * 16 + s) * CF
    _fill(msg0, CHUNK, C, 0.0)
    row0 = s * ROWS_PER_TILE
    def zero_body(k, _):
        pltpu.sync_copy(msg0, acc_sp.at[pl.ds(row0 + k * CHUNK, CHUNK)])
        return 0
    lax.fori_loop(0, ROWS_PER_TILE // CHUNK, zero_body, 0)
    pltpu.sync_copy(pk_hbm.at[pl.ds(base * CHUNK, CF * CHUNK)], pidx)
    plsc.subcore_barrier()

    def unpack(ch, ia, idd):
        for g in range(C // 16):
            w = pidx[pl.ds(ch * CHUNK + g * 16, 16)]
            ia[pl.ds(g * 16, 16)] = lax.bitwise_and(w, 0xFFFF)
            idd[pl.ds(g * 16, 16)] = lax.shift_right_logical(w, 16)

    def wait_gather(buf, sem):
        pltpu.make_async_copy(y_hbm.at[ia0], buf, sem).wait()

    def wait_scat(buf, sem):
        pltpu.make_async_copy(buf, acc_sp.at[id0], sem).wait()

    unpack(0, ia0, id0)
    pltpu.async_copy(y_hbm.at[ia0], msg0, gsem0)
    unpack(1, ia1, id1)
    pltpu.async_copy(y_hbm.at[ia1], msg1, gsem1)
    def body(j, _):
        a = 2 * j
        b = 2 * j + 1
        wait_gather(msg0, gsem0)
        pltpu.async_copy(msg0, acc_sp.at[id0], ssem0, add=True)
        wait_gather(msg1, gsem1)
        pltpu.async_copy(msg1, acc_sp.at[id1], ssem1, add=True)
        wait_scat(msg0, ssem0)
        unpack(a + 2, ia0, id0)
        pltpu.async_copy(y_hbm.at[ia0], msg0, gsem0)
        wait_scat(msg1, ssem1)
        unpack(b + 2, ia1, id1)
        pltpu.async_copy(y_hbm.at[ia1], msg1, gsem1)
        return 0
    lax.fori_loop(0, CF // 2 - 1, body, 0)
    wait_gather(msg0, gsem0)
    pltpu.async_copy(msg0, acc_sp.at[id0], ssem0, add=True)
    wait_gather(msg1, gsem1)
    pltpu.async_copy(msg1, acc_sp.at[id1], ssem1, add=True)
    wait_scat(msg0, ssem0)
    wait_scat(msg1, ssem1)
    plsc.subcore_barrier()
    pltpu.sync_copy(
        acc_sp.at[pl.ds(row0, ROWS_PER_TILE)],
        out_hbm.at[pl.ds(c * NROWS + row0, ROWS_PER_TILE)],
    )


_BS = 1024
_GRID = NROWS // _BS


def _tc1_body(d0_ref, d1_ref, x_ref, w_ref, y_ref, dinv_ref):
    deg = d0_ref[...] + d1_ref[...] + 1.0
    dinv8 = lax.rsqrt(deg)
    nrow = _BS // CHUNK
    i0 = lax.broadcasted_iota(jnp.int32, (_BS, nrow), 0)
    i1 = lax.broadcasted_iota(jnp.int32, (_BS, nrow), 1)
    rowsel = (i0 // CHUNK == i1).astype(jnp.float32)
    t = jnp.dot(rowsel, dinv8, preferred_element_type=jnp.float32)
    j0 = lax.broadcasted_iota(jnp.int32, (_BS, CHUNK), 0)
    j1 = lax.broadcasted_iota(jnp.int32, (_BS, CHUNK), 1)
    lanemask = (j1 == j0 % CHUNK).astype(jnp.float32)
    dinv = jnp.sum(t * lanemask, axis=1, keepdims=True)
    xw = jnp.dot(x_ref[...], w_ref[...], preferred_element_type=jnp.float32)
    y_ref[...] = xw * dinv
    dinv_ref[...] = jnp.broadcast_to(dinv, (_BS, 16))


def _tc2_body(p0_ref, p1_ref, y1_ref, dinv_ref, w_ref, b1_ref, y2_ref):
    dv = dinv_ref[:, 0:1]
    agg = p0_ref[...] + p1_ref[...] + y1_ref[...]
    h = jnp.maximum(agg * dv + b1_ref[...], 0.0)
    y2_ref[...] = jnp.dot(h, w_ref[...], preferred_element_type=jnp.float32) * dv


def _tc3_body(q0_ref, q1_ref, y2_ref, dinv_ref, b2_ref, out_ref):
    dv = dinv_ref[:, 0:1]
    out_ref[...] = (q0_ref[...] + q1_ref[...] + y2_ref[...]) * dv + b2_ref[...]


def _rows(i):
    return (i, 0)


def _rows_hi(i):
    return (i + _GRID, 0)


def _rep(i):
    return (0, 0)


def kernel(x, edge_index, W1, b1, W2, b2):
    ei = edge_index.astype(jnp.int32)
    src = jnp.concatenate(
        [ei[0], jnp.zeros((E_ALLOC - N_EDGES,), jnp.int32)])
    pad_dst = GROW + jnp.arange(E_ALLOC - N_EDGES, dtype=jnp.int32) % (NROWS - GROW)
    dst = jnp.concatenate([ei[1], pad_dst])
    packed = src | (dst << 16)
    packed2d = packed[:E_PAD].reshape(-1, CHUNK)
    x_pad = jnp.concatenate(
        [x, jnp.zeros((NROWS - N_NODES, C), jnp.float32)], axis=0)
    b1r = b1.reshape(1, C)
    b2r = b2.reshape(1, C)

    degp = _sc_degree(packed2d)

    y1, dinv = pl.pallas_call(
        _tc1_body,
        grid=(_GRID,),
        in_specs=[
            pl.BlockSpec((_BS // CHUNK, CHUNK), _rows),
            pl.BlockSpec((_BS // CHUNK, CHUNK), _rows_hi),
            pl.BlockSpec((_BS, C), _rows),
            pl.BlockSpec((C, C), _rep),
        ],
        out_specs=[
            pl.BlockSpec((_BS, C), _rows),
            pl.BlockSpec((_BS, 16), _rows),
        ],
        out_shape=[
            jax.ShapeDtypeStruct((NROWS, C), jnp.float32),
            jax.ShapeDtypeStruct((NROWS, 16), jnp.float32),
        ],
    )(degp, degp, x_pad, W1)

    p = _sc_segsum(y1, packed)

    y2 = pl.pallas_call(
        _tc2_body,
        grid=(_GRID,),
        in_specs=[
            pl.BlockSpec((_BS, C), _rows),
            pl.BlockSpec((_BS, C), _rows_hi),
            pl.BlockSpec((_BS, C), _rows),
            pl.BlockSpec((_BS, 16), _rows),
            pl.BlockSpec((C, C), _rep),
            pl.BlockSpec((1, C), _rep),
        ],
        out_specs=pl.BlockSpec((_BS, C), _rows),
        out_shape=jax.ShapeDtypeStruct((NROWS, C), jnp.float32),
    )(p, p, y1, dinv, W2, b1r)

    q = _sc_segsum(y2, packed)

    out = pl.pallas_call(
        _tc3_body,
        grid=(_GRID,),
        in_specs=[
            pl.BlockSpec((_BS, C), _rows),
            pl.BlockSpec((_BS, C), _rows_hi),
            pl.BlockSpec((_BS, C), _rows),
            pl.BlockSpec((_BS, 16), _rows),
            pl.BlockSpec((1, C), _rep),
        ],
        out_specs=pl.BlockSpec((_BS, C), _rows),
        out_shape=jax.ShapeDtypeStruct((NROWS, C), jnp.float32),
    )(q, q, y2, dinv, b2r)

    return out[:N_NODES]
